# baseline (speedup 1.0000x reference)
import functools

import jax
import jax.numpy as jnp
from jax import lax
from jax.experimental import pallas as pl
from jax.experimental.pallas import tpu as pltpu
from jax.experimental.pallas import tpu_sc as plsc

B = 32
H = 768
NB = 2000

NC = 2
NS = 16
NW = NC * NS
ROWS_PER_BATCH = 200 + 5 * 15 + 3 + 2
CHUNK = 56
NCHUNK = ROWS_PER_BATCH // CHUNK


def _sc_gather_body(table_hbm, idx_hbm, out_hbm, idx_v, rows_v, sem):
    wid = lax.axis_index("s") * NC + lax.axis_index("c")
    base = wid * ROWS_PER_BATCH
    pltpu.sync_copy(idx_hbm.at[pl.ds(base, ROWS_PER_BATCH)], idx_v)
    for c in range(NCHUNK):
        pltpu.async_copy(table_hbm.at[idx_v.at[pl.ds(c * CHUNK, CHUNK)]],
                         rows_v, sem).wait()
        pltpu.sync_copy(rows_v, out_hbm.at[pl.ds(base + c * CHUNK, CHUNK)])


_sc_gather = functools.partial(
    pl.kernel,
    _sc_gather_body,
    out_type=jax.ShapeDtypeStruct((B * ROWS_PER_BATCH, H), jnp.float32),
    mesh=plsc.VectorSubcoreMesh(core_axis_name="c", subcore_axis_name="s",
                                num_cores=NC, num_subcores=NS),
    scratch_types=[
        pltpu.VMEM((ROWS_PER_BATCH,), jnp.int32),
        pltpu.VMEM((CHUNK, H), jnp.float32),
        pltpu.SemaphoreType.DMA,
    ],
)()


def _tc_body(img_ref, lab_num_ref, stag_ref, micro_num_ref, micro_com_ref,
             med_ref, fam_ref, pat_num_ref, tri_num_ref, chief_ref,
             Wlab_ref, blab_ref, Wmic_ref, bmic_ref,
             Wage_ref, bage_ref, Wtri_ref, btri_ref,
             out_ref):
    relu = lambda x: jnp.maximum(x, 0.0)
    bf = jnp.bfloat16
    stag = stag_ref[0]

    out_ref[0, 0:256] = img_ref[0]

    val = relu(jnp.dot(lab_num_ref[0].astype(bf), Wlab_ref[:].astype(bf),
                       preferred_element_type=jnp.float32) + blab_ref[:])
    ids = stag[0:200]
    out_ref[0, 256:656] = jnp.concatenate([val, ids], axis=1).reshape(400, H)

    dil_val = relu(jnp.dot(micro_num_ref[0].astype(bf), Wmic_ref[:].astype(bf),
                           preferred_element_type=jnp.float32) + bmic_ref[:])
    micro = jnp.concatenate(
        [stag[200:215], stag[215:230], stag[230:245], stag[245:260],
         stag[260:275], dil_val, micro_com_ref[0]],
        axis=1).reshape(105, H)
    out_ref[0, 656:761] = micro

    out_ref[0, 761:889] = med_ref[0]
    out_ref[0, 889:953] = fam_ref[0]

    age = relu(jnp.dot(pat_num_ref[0].astype(bf), Wage_ref[:].astype(bf),
                       preferred_element_type=jnp.float32) + bage_ref[:])
    out_ref[0, 953:956] = stag[275:278]
    out_ref[0, 956:957] = age

    vit = relu(jnp.dot(tri_num_ref[0].astype(bf), Wtri_ref[:].astype(bf),
                       preferred_element_type=jnp.float32) + btri_ref[:])
    out_ref[0, 957:963] = vit
    out_ref[0, 963:965] = stag[278:280]

    out_ref[0, 965:997] = chief_ref[0]


def kernel(image_feature, labevent_number_input, labevent_category_input,
           microbiology_category_input, microbiology_number_input,
           microbiology_comment_embeddings, medical_history_embeddings,
           family_history_embeddings, patient_category_input,
           patient_number_input, triage_category_input, triage_number_input,
           chiefcomplaint_embedding, total_attention_mask,
           multimodal_input_type, labevent_table, spec_table, test_table,
           org_table, ab_table, dil_table, patient_table, triage_table,
           W_lab, b_lab, W_micro, b_micro, W_age, b_age, W_triage, b_triage):
    i32 = jnp.int32

    combined_table = jnp.concatenate(
        [labevent_table, spec_table, test_table, org_table, ab_table,
         dil_table, patient_table, triage_table], axis=0)
    mc = microbiology_category_input.astype(i32)
    combined_idx = jnp.concatenate(
        [labevent_category_input.astype(i32),
         1000 + mc[:, 0::5],
         1200 + mc[:, 1::5],
         1400 + mc[:, 2::5],
         2200 + mc[:, 3::5],
         2300 + mc[:, 4::5],
         2316 + patient_category_input.astype(i32),
         2380 + triage_category_input[:, -2:].astype(i32),
         ], axis=1).reshape(B * ROWS_PER_BATCH)

    staging = _sc_gather(combined_table, combined_idx)
    staging = staging.reshape(B, ROWS_PER_BATCH, H)

    def batch_spec(shape):
        nd = len(shape)
        return pl.BlockSpec((1,) + shape, lambda b: (b,) + (0,) * nd)

    def const_spec(shape):
        return pl.BlockSpec(shape, lambda b: (0,) * len(shape))

    in_specs = [
        batch_spec((256, H)),
        batch_spec((200, NB)),
        batch_spec((ROWS_PER_BATCH, H)),
        batch_spec((15, NB)),
        batch_spec((15, H)),
        batch_spec((128, H)),
        batch_spec((64, H)),
        batch_spec((1, NB)),
        batch_spec((6, NB)),
        batch_spec((32, H)),
        const_spec((NB, H)),
        const_spec((1, H)),
        const_spec((NB, H)),
        const_spec((1, H)),
        const_spec((NB, H)),
        const_spec((1, H)),
        const_spec((NB, H)),
        const_spec((1, H)),
    ]

    out = pl.pallas_call(
        _tc_body,
        grid=(B,),
        in_specs=in_specs,
        out_specs=pl.BlockSpec((1, 997, H), lambda b: (b, 0, 0)),
        out_shape=jax.ShapeDtypeStruct((B, 997, H), jnp.float32),
    )(
        image_feature, labevent_number_input, staging,
        microbiology_number_input, microbiology_comment_embeddings,
        medical_history_embeddings, family_history_embeddings,
        patient_number_input.astype(jnp.float32),
        triage_number_input.astype(jnp.float32),
        chiefcomplaint_embedding,
        W_lab, b_lab.reshape(1, H), W_micro, b_micro.reshape(1, H),
        W_age, b_age.reshape(1, H), W_triage, b_triage.reshape(1, H),
    )
    return out


# speedup vs baseline: 1.1885x; 1.1885x over previous
import functools

import jax
import jax.numpy as jnp
from jax import lax
from jax.experimental import pallas as pl
from jax.experimental.pallas import tpu as pltpu
from jax.experimental.pallas import tpu_sc as plsc

B = 32
H = 768
NB = 2000

NC = 2
NS = 16
ROWS_PER_BATCH = 200
CHUNKS = ((0, 104), (104, 96))


def _sc_gather_body(table_hbm, idx_hbm, out_hbm, idx_v, rows_v, sem):
    wid = lax.axis_index("s") * NC + lax.axis_index("c")
    base = wid * ROWS_PER_BATCH
    pltpu.sync_copy(idx_hbm.at[pl.ds(base, ROWS_PER_BATCH)], idx_v)
    for off, cnt in CHUNKS:
        pltpu.async_copy(table_hbm.at[idx_v.at[pl.ds(off, cnt)]],
                         rows_v.at[pl.ds(0, cnt)], sem).wait()
        pltpu.sync_copy(rows_v.at[pl.ds(0, cnt)],
                        out_hbm.at[pl.ds(base + off, cnt)])


_sc_gather = functools.partial(
    pl.kernel,
    _sc_gather_body,
    out_type=jax.ShapeDtypeStruct((B * ROWS_PER_BATCH, H), jnp.float32),
    mesh=plsc.VectorSubcoreMesh(core_axis_name="c", subcore_axis_name="s",
                                num_cores=NC, num_subcores=NS),
    scratch_types=[
        pltpu.VMEM((ROWS_PER_BATCH,), jnp.int32),
        pltpu.VMEM((104, H), jnp.float32),
        pltpu.SemaphoreType.DMA,
    ],
)()


def _onehot_gather(table, idx, n):
    oh = (jax.lax.broadcasted_iota(jnp.int32, (idx.shape[0], n), 1)
          == idx[:, None]).astype(jnp.float32)
    return jnp.dot(oh, table, preferred_element_type=jnp.float32)


def _tc_body(img_ref, lab_num_ref, stag_ref,
             spec_idx_ref, test_idx_ref, org_idx_ref, ab_idx_ref,
             dil_idx_ref, micro_num_ref, micro_com_ref, med_ref, fam_ref,
             pat_idx_ref, pat_num_ref, tri_idx_ref, tri_num_ref, chief_ref,
             spec_tab_ref, test_tab_ref, org_tab_ref, ab_tab_ref,
             dil_tab_ref, pat_tab_ref, tri_tab_ref,
             Wlab_ref, blab_ref, Wmic_ref, bmic_ref,
             Wage_ref, bage_ref, Wtri_ref, btri_ref,
             out_ref):
    relu = lambda x: jnp.maximum(x, 0.0)

    out_ref[0, 0:256] = img_ref[0]

    val = relu(jnp.dot(lab_num_ref[0], Wlab_ref[:],
                       preferred_element_type=jnp.float32) + blab_ref[:])
    ids = stag_ref[0]
    out_ref[0, 256:656] = jnp.concatenate([val, ids], axis=1).reshape(400, H)

    spec_f = _onehot_gather(spec_tab_ref[:], spec_idx_ref[0, 0], 200)
    test_f = _onehot_gather(test_tab_ref[:], test_idx_ref[0, 0], 200)
    org_f = _onehot_gather(org_tab_ref[:], org_idx_ref[0, 0], 800)
    ab_f = _onehot_gather(ab_tab_ref[:], ab_idx_ref[0, 0], 100)
    dil_f = _onehot_gather(dil_tab_ref[:], dil_idx_ref[0, 0], 16)
    dil_val = relu(jnp.dot(micro_num_ref[0], Wmic_ref[:],
                           preferred_element_type=jnp.float32) + bmic_ref[:])
    micro = jnp.concatenate(
        [spec_f, test_f, org_f, ab_f, dil_f, dil_val, micro_com_ref[0]],
        axis=1).reshape(105, H)
    out_ref[0, 656:761] = micro

    out_ref[0, 761:889] = med_ref[0]
    out_ref[0, 889:953] = fam_ref[0]

    pat_f = _onehot_gather(pat_tab_ref[:], pat_idx_ref[0, 0], 64)
    age = relu(jnp.dot(pat_num_ref[0], Wage_ref[:],
                       preferred_element_type=jnp.float32) + bage_ref[:])
    out_ref[0, 953:956] = pat_f
    out_ref[0, 956:957] = age

    vit = relu(jnp.dot(tri_num_ref[0], Wtri_ref[:],
                       preferred_element_type=jnp.float32) + btri_ref[:])
    pa_f = _onehot_gather(tri_tab_ref[:], tri_idx_ref[0, 0], 32)
    out_ref[0, 957:963] = vit
    out_ref[0, 963:965] = pa_f

    out_ref[0, 965:997] = chief_ref[0]


def kernel(image_feature, labevent_number_input, labevent_category_input,
           microbiology_category_input, microbiology_number_input,
           microbiology_comment_embeddings, medical_history_embeddings,
           family_history_embeddings, patient_category_input,
           patient_number_input, triage_category_input, triage_number_input,
           chiefcomplaint_embedding, total_attention_mask,
           multimodal_input_type, labevent_table, spec_table, test_table,
           org_table, ab_table, dil_table, patient_table, triage_table,
           W_lab, b_lab, W_micro, b_micro, W_age, b_age, W_triage, b_triage):
    i32 = jnp.int32

    lab_idx_flat = labevent_category_input.astype(i32).reshape(
        B * ROWS_PER_BATCH)
    staging = _sc_gather(labevent_table, lab_idx_flat)
    staging = staging.reshape(B, ROWS_PER_BATCH, H)

    spec_idx = microbiology_category_input[:, 0::5].astype(i32).reshape(B, 1, 15)
    test_idx = microbiology_category_input[:, 1::5].astype(i32).reshape(B, 1, 15)
    org_idx = microbiology_category_input[:, 2::5].astype(i32).reshape(B, 1, 15)
    ab_idx = microbiology_category_input[:, 3::5].astype(i32).reshape(B, 1, 15)
    dil_idx = microbiology_category_input[:, 4::5].astype(i32).reshape(B, 1, 15)
    pat_idx = patient_category_input.astype(i32).reshape(B, 1, 3)
    tri_idx = triage_category_input[:, -2:].astype(i32).reshape(B, 1, 2)

    def batch_spec(shape):
        nd = len(shape)
        return pl.BlockSpec((1,) + shape, lambda b: (b,) + (0,) * nd)

    def const_spec(shape):
        return pl.BlockSpec(shape, lambda b: (0,) * len(shape))

    in_specs = [
        batch_spec((256, H)),
        batch_spec((200, NB)),
        batch_spec((ROWS_PER_BATCH, H)),
        batch_spec((1, 15)),
        batch_spec((1, 15)),
        batch_spec((1, 15)),
        batch_spec((1, 15)),
        batch_spec((1, 15)),
        batch_spec((15, NB)),
        batch_spec((15, H)),
        batch_spec((128, H)),
        batch_spec((64, H)),
        batch_spec((1, 3)),
        batch_spec((1, NB)),
        batch_spec((1, 2)),
        batch_spec((6, NB)),
        batch_spec((32, H)),
        const_spec((200, H)),
        const_spec((200, H)),
        const_spec((800, H)),
        const_spec((100, H)),
        const_spec((16, H)),
        const_spec((64, H)),
        const_spec((32, H)),
        const_spec((NB, H)),
        const_spec((1, H)),
        const_spec((NB, H)),
        const_spec((1, H)),
        const_spec((NB, H)),
        const_spec((1, H)),
        const_spec((NB, H)),
        const_spec((1, H)),
    ]

    out = pl.pallas_call(
        _tc_body,
        grid=(B,),
        in_specs=in_specs,
        out_specs=pl.BlockSpec((1, 997, H), lambda b: (b, 0, 0)),
        out_shape=jax.ShapeDtypeStruct((B, 997, H), jnp.float32),
    )(
        image_feature, labevent_number_input, staging,
        spec_idx, test_idx, org_idx, ab_idx, dil_idx,
        microbiology_number_input, microbiology_comment_embeddings,
        medical_history_embeddings, family_history_embeddings,
        pat_idx, patient_number_input.astype(jnp.float32),
        tri_idx, triage_number_input.astype(jnp.float32),
        chiefcomplaint_embedding,
        spec_table, test_table, org_table, ab_table, dil_table,
        patient_table, triage_table,
        W_lab, b_lab.reshape(1, H), W_micro, b_micro.reshape(1, H),
        W_age, b_age.reshape(1, H), W_triage, b_triage.reshape(1, H),
    )
    return out
